# baseline (speedup 1.0000x reference)
import jax
import jax.numpy as jnp
from jax.experimental import pallas as pl

N = 4096
D = 512
BM = 1024


def _body(adj_ref, x_ref, w4_ref, w8_ref, w16_ref, w32_ref, o_ref):
    a16 = adj_ref[...].astype(jnp.bfloat16)
    x16 = x_ref[...].astype(jnp.bfloat16)
    y = jnp.dot(a16, x16, preferred_element_type=jnp.float32)
    y16 = y.astype(jnp.bfloat16)
    def m(w_ref):
        w16 = w_ref[...].astype(jnp.bfloat16)
        return jnp.maximum(jnp.dot(y16, w16, preferred_element_type=jnp.float32), 0.0)
    o_ref[...] = m(w4_ref) + m(w8_ref) + m(w16_ref) + m(w32_ref)


@jax.jit
def _run(x, adj, W4, W8, W16, W32):
    grid = (N // BM,)
    w_spec = pl.BlockSpec((D, D), lambda i: (0, 0))
    return pl.pallas_call(
        _body,
        grid=grid,
        in_specs=[
            pl.BlockSpec((BM, N), lambda i: (i, 0)),
            pl.BlockSpec((N, D), lambda i: (0, 0)),
            w_spec, w_spec, w_spec, w_spec,
        ],
        out_specs=pl.BlockSpec((BM, D), lambda i: (i, 0)),
        out_shape=jax.ShapeDtypeStruct((N, D), jnp.float32),
    )(adj, x, W4, W8, W16, W32)


def kernel(x, adj, now_epoch, W4, W8, W16, W32):
    return _run(x, adj, W4, W8, W16, W32)


# speedup vs baseline: 1.0199x; 1.0199x over previous
import jax
import jax.numpy as jnp
from jax.experimental import pallas as pl

N = 4096
D = 512
BM = 512


def _body(adj_ref, x_ref, w4_ref, w8_ref, w16_ref, w32_ref, o_ref):
    a16 = adj_ref[...].astype(jnp.bfloat16)
    x16 = x_ref[...].astype(jnp.bfloat16)
    y = jnp.dot(a16, x16, preferred_element_type=jnp.float32)
    y16 = y.astype(jnp.bfloat16)
    def m(w_ref):
        w16 = w_ref[...].astype(jnp.bfloat16)
        return jnp.maximum(jnp.dot(y16, w16, preferred_element_type=jnp.float32), 0.0)
    o_ref[...] = m(w4_ref) + m(w8_ref) + m(w16_ref) + m(w32_ref)


@jax.jit
def _run(x, adj, W4, W8, W16, W32):
    grid = (N // BM,)
    w_spec = pl.BlockSpec((D, D), lambda i: (0, 0))
    return pl.pallas_call(
        _body,
        grid=grid,
        in_specs=[
            pl.BlockSpec((BM, N), lambda i: (i, 0)),
            pl.BlockSpec((N, D), lambda i: (0, 0)),
            w_spec, w_spec, w_spec, w_spec,
        ],
        out_specs=pl.BlockSpec((BM, D), lambda i: (i, 0)),
        out_shape=jax.ShapeDtypeStruct((N, D), jnp.float32),
    )(adj, x, W4, W8, W16, W32)


def kernel(x, adj, now_epoch, W4, W8, W16, W32):
    return _run(x, adj, W4, W8, W16, W32)
